# Initial kernel scaffold; baseline (speedup 1.0000x reference)
import jax
import jax.numpy as jnp
from jax.experimental import pallas as pl


def kernel(x, edge_index, W1, b1, beta2, beta3, W4, b4):
    raise NotImplementedError("write your pallas kernel here")


# speedup vs baseline: 9.0070x; 9.0070x over previous
import functools

import jax
import jax.numpy as jnp
from jax import lax
from jax.experimental import pallas as pl
from jax.experimental.pallas import tpu as pltpu, tpu_sc as plsc

NC = 2
NS = 16
NW = NC * NS
CK = 128


def _dense1_body(x_ref, w_ref, b_ref, o_ref):
    h = jnp.dot(x_ref[...], w_ref[...], preferred_element_type=jnp.float32)
    o_ref[...] = jnp.maximum(h + b_ref[...], 0.0)


def _dense1(x, W1, b1):
    n, d = x.shape
    h_ = W1.shape[1]
    blk = 2000
    return pl.pallas_call(
        _dense1_body,
        grid=(n // blk,),
        in_specs=[
            pl.BlockSpec((blk, d), lambda i: (i, 0)),
            pl.BlockSpec((d, h_), lambda i: (0, 0)),
            pl.BlockSpec((1, h_), lambda i: (0, 0)),
        ],
        out_specs=pl.BlockSpec((blk, h_), lambda i: (i, 0)),
        out_shape=jax.ShapeDtypeStruct((n, h_), jnp.float32),
    )(x, W1, b1.reshape(1, h_))


def _make_gather(n_pad, ep, h_):
    ch = ep // (NW * CK)
    t = ch * CK
    mesh = plsc.VectorSubcoreMesh(core_axis_name="c", subcore_axis_name="s")

    @functools.partial(
        pl.kernel,
        mesh=mesh,
        out_type=(
            jax.ShapeDtypeStruct((ep, h_), jnp.float32),
            jax.ShapeDtypeStruct((ep, h_), jnp.float32),
        ),
        scratch_types=[
            pltpu.VMEM((ch, CK), jnp.int32),
            pltpu.VMEM((ch, CK), jnp.int32),
            pltpu.VMEM((CK, h_), jnp.float32),
            pltpu.VMEM((CK, h_), jnp.float32),
            pltpu.SemaphoreType.DMA,
            pltpu.SemaphoreType.DMA,
        ],
        compiler_params=pltpu.CompilerParams(use_tc_tiling_on_sc=False),
    )
    def gather_k(h_hbm, src_hbm, dst_hbm, hs_out, hd_out,
                 sidx, didx, bufs, bufd, gs_sem, gd_sem):
        cid = lax.axis_index("c")
        sid = lax.axis_index("s")
        wid = sid * NC + cid
        row0 = wid * ch

        pltpu.sync_copy(src_hbm.at[pl.ds(row0, ch)], sidx)
        pltpu.sync_copy(dst_hbm.at[pl.ds(row0, ch)], didx)

        def body(j, carry):
            cs = pltpu.async_copy(h_hbm.at[sidx.at[j]], bufs, gs_sem)
            cd = pltpu.async_copy(h_hbm.at[didx.at[j]], bufd, gd_sem)
            cs.wait()
            cd.wait()
            base = wid * t + j * CK
            pltpu.sync_copy(bufs, hs_out.at[pl.ds(base, CK)])
            pltpu.sync_copy(bufd, hd_out.at[pl.ds(base, CK)])
            return carry

        lax.fori_loop(0, ch, body, 0, unroll=False)

    return gather_k


def _edge_body(hs_ref, hd_ref, beta_ref, c_ref, s_ref):
    hs = hs_ref[...]
    hd = hd_ref[...]
    beta = beta_ref[0, 0]
    dot = jnp.sum(hs * hd, axis=1)
    ns = jnp.maximum(jnp.sqrt(jnp.sum(hs * hs, axis=1)), 1e-12)
    nd = jnp.maximum(jnp.sqrt(jnp.sum(hd * hd, axis=1)), 1e-12)
    s = jnp.exp(beta * (dot / (ns * nd)))
    c_ref[...] = s[:, None] * hs
    s_ref[...] = s


def _edge(hs, hd, beta):
    ep, h_ = hs.shape
    blk = 4096
    return pl.pallas_call(
        _edge_body,
        grid=(ep // blk,),
        in_specs=[
            pl.BlockSpec((blk, h_), lambda i: (i, 0)),
            pl.BlockSpec((blk, h_), lambda i: (i, 0)),
            pl.BlockSpec((1, 1), lambda i: (0, 0)),
        ],
        out_specs=[
            pl.BlockSpec((blk, h_), lambda i: (i, 0)),
            pl.BlockSpec((blk,), lambda i: (i,)),
        ],
        out_shape=[
            jax.ShapeDtypeStruct((ep, h_), jnp.float32),
            jax.ShapeDtypeStruct((ep,), jnp.float32),
        ],
    )(hs, hd, beta.reshape(1, 1))


def _make_scatter(n_pad, ep, h_):
    ch = ep // (NW * CK)
    t = ch * CK
    rows_per = n_pad // NS
    zc = rows_per // CK
    mesh = plsc.VectorSubcoreMesh(core_axis_name="c", subcore_axis_name="s")

    @functools.partial(
        pl.kernel,
        mesh=mesh,
        out_type=(
            jax.ShapeDtypeStruct((NC, n_pad, h_), jnp.float32),
            jax.ShapeDtypeStruct((NC, n_pad), jnp.float32),
        ),
        scratch_types=[
            pltpu.VMEM((ch, CK), jnp.int32),
            pltpu.VMEM((CK, h_), jnp.float32),
            pltpu.VMEM((CK,), jnp.float32),
            pltpu.VMEM_SHARED((n_pad, h_), jnp.float32),
            pltpu.VMEM_SHARED((n_pad,), jnp.float32),
        ],
        compiler_params=pltpu.CompilerParams(use_tc_tiling_on_sc=False),
    )
    def scatter_k(contrib_hbm, sval_hbm, dst_hbm, z16_hbm, z1_hbm,
                  acc_out, den_out, didx, cbuf, dbuf, acc_sh, den_sh):
        cid = lax.axis_index("c")
        sid = lax.axis_index("s")
        wid = sid * NC + cid

        r0 = sid * rows_per
        pltpu.sync_copy(z16_hbm.at[pl.ds(r0, rows_per)], acc_sh.at[pl.ds(r0, rows_per)])
        pltpu.sync_copy(z1_hbm.at[pl.ds(r0, rows_per)], den_sh.at[pl.ds(r0, rows_per)])
        plsc.subcore_barrier()

        pltpu.sync_copy(dst_hbm.at[pl.ds(wid * ch, ch)], didx)

        def body(j, carry):
            base = wid * t + j * CK
            pltpu.sync_copy(contrib_hbm.at[pl.ds(base, CK)], cbuf)
            pltpu.sync_copy(sval_hbm.at[wid * ch + j], dbuf)
            pltpu.sync_copy(cbuf, acc_sh.at[didx.at[j]], add=True)
            pltpu.sync_copy(dbuf, den_sh.at[didx.at[j]], add=True)
            return carry

        lax.fori_loop(0, ch, body, 0, unroll=False)
        plsc.subcore_barrier()

        def wb(k, carry):
            r = sid * rows_per + k * CK
            pltpu.sync_copy(acc_sh.at[pl.ds(r, CK)], cbuf)
            pltpu.sync_copy(cbuf, acc_out.at[cid].at[pl.ds(r, CK)])
            pltpu.sync_copy(den_sh.at[pl.ds(r, CK)], dbuf)
            pltpu.sync_copy(dbuf, den_out.at[cid].at[pl.ds(r, CK)])
            return carry

        lax.fori_loop(0, zc, wb, 0, unroll=False)

    return scatter_k


def _combine_body(acc_ref, den_ref, o_ref):
    a = acc_ref[0] + acc_ref[1]
    d = jnp.maximum(den_ref[0] + den_ref[1], 1e-30)
    o_ref[...] = a / d


def _combine(acc, den):
    _, n_pad, h_ = acc.shape
    blk = 2048
    return pl.pallas_call(
        _combine_body,
        grid=(n_pad // blk,),
        in_specs=[
            pl.BlockSpec((2, blk, h_), lambda i: (0, i, 0)),
            pl.BlockSpec((2, blk, 1), lambda i: (0, i, 0)),
        ],
        out_specs=pl.BlockSpec((blk, h_), lambda i: (i, 0)),
        out_shape=jax.ShapeDtypeStruct((n_pad, h_), jnp.float32),
    )(acc, den)


def _final_body(acc_ref, den_ref, w_ref, b_ref, o_ref):
    a = acc_ref[0] + acc_ref[1]
    d = jnp.maximum(den_ref[0] + den_ref[1], 1e-30)
    h = a / d
    logits = jnp.dot(h, w_ref[...], preferred_element_type=jnp.float32) + b_ref[...]
    m = jnp.max(logits, axis=1, keepdims=True)
    lse = m + jnp.log(jnp.sum(jnp.exp(logits - m), axis=1, keepdims=True))
    o_ref[...] = logits - lse


def _final(acc, den, W4, b4, n):
    _, n_pad, h_ = acc.shape
    c = W4.shape[1]
    blk = 2000
    return pl.pallas_call(
        _final_body,
        grid=(n // blk,),
        in_specs=[
            pl.BlockSpec((2, blk, h_), lambda i: (0, i, 0)),
            pl.BlockSpec((2, blk, 1), lambda i: (0, i, 0)),
            pl.BlockSpec((h_, c), lambda i: (0, 0)),
            pl.BlockSpec((1, c), lambda i: (0, 0)),
        ],
        out_specs=pl.BlockSpec((blk, c), lambda i: (i, 0)),
        out_shape=jax.ShapeDtypeStruct((n, c), jnp.float32),
    )(acc, den, W4, b4.reshape(1, c))


def kernel(x, edge_index, W1, b1, beta2, beta3, W4, b4):
    n, d = x.shape
    h_ = W1.shape[1]
    e = edge_index.shape[1]

    n_pad = 51200
    etot = e + n
    sup = NW * CK
    ep = ((etot + sup - 1) // sup) * sup
    pad = ep - etot

    loops = jnp.arange(n, dtype=jnp.int32)
    junk = jnp.full((pad,), n, dtype=jnp.int32)
    src = jnp.concatenate([edge_index[0], loops, junk]).reshape(ep // CK, CK)
    dst = jnp.concatenate([edge_index[1], loops, junk]).reshape(ep // CK, CK)

    h1 = _dense1(x, W1, b1)
    h1p = jnp.concatenate([h1, jnp.zeros((n_pad - n, h_), jnp.float32)], axis=0)

    z16 = jnp.zeros((n_pad, h_), jnp.float32)
    z1 = jnp.zeros((n_pad,), jnp.float32)

    gather_k = _make_gather(n_pad, ep, h_)
    scatter_k = _make_scatter(n_pad, ep, h_)

    def conv(hp, beta):
        hs, hd = gather_k(hp, src, dst)
        contrib, sval = _edge(hs, hd, beta)
        acc, den = scatter_k(contrib, sval.reshape(ep // CK, CK), dst, z16, z1)
        return acc, den.reshape(NC, n_pad, 1)

    acc1, den1 = conv(h1p, beta2)
    h2p = _combine(acc1, den1)
    acc2, den2 = conv(h2p, beta3)
    return _final(acc2, den2, W4, b4, n)


# speedup vs baseline: 43.1461x; 4.7903x over previous
import functools

import jax
import jax.numpy as jnp
from jax import lax
from jax.experimental import pallas as pl
from jax.experimental.pallas import tpu as pltpu, tpu_sc as plsc

NC = 2
NS = 16
NW = NC * NS
CK = 128


def _dense1_body(x_ref, w_ref, b_ref, o_ref):
    h = jnp.dot(x_ref[...], w_ref[...], preferred_element_type=jnp.float32)
    o_ref[...] = jnp.maximum(h + b_ref[...], 0.0)


def _dense1(x, W1, b1):
    n, d = x.shape
    h_ = W1.shape[1]
    blk = 2000
    return pl.pallas_call(
        _dense1_body,
        grid=(n // blk,),
        in_specs=[
            pl.BlockSpec((blk, d), lambda i: (i, 0)),
            pl.BlockSpec((d, h_), lambda i: (0, 0)),
            pl.BlockSpec((1, h_), lambda i: (0, 0)),
        ],
        out_specs=pl.BlockSpec((blk, h_), lambda i: (i, 0)),
        out_shape=jax.ShapeDtypeStruct((n, h_), jnp.float32),
    )(x, W1, b1.reshape(1, h_))


def _rsqrt16(v):
    i = plsc.bitcast(v, jnp.int32)
    y = plsc.bitcast(jnp.int32(0x5F3759DF) - (i >> 1), jnp.float32)
    for _ in range(3):
        y = y * (1.5 - 0.5 * v * y * y)
    return y


def _make_conv(n_pad, ep, h_):
    ch = ep // (NW * CK)
    rows_per = n_pad // NS
    zc = rows_per // CK
    ng = CK // 16
    mesh = plsc.VectorSubcoreMesh(core_axis_name="c", subcore_axis_name="s")

    @functools.partial(
        pl.kernel,
        mesh=mesh,
        out_type=(
            jax.ShapeDtypeStruct((NC, n_pad, h_), jnp.float32),
            jax.ShapeDtypeStruct((NC, n_pad), jnp.float32),
        ),
        scratch_types=[
            pltpu.VMEM((ch, CK), jnp.int32),
            pltpu.VMEM((ch, CK), jnp.int32),
            pltpu.VMEM((CK, h_), jnp.float32),
            pltpu.VMEM((CK, h_), jnp.float32),
            pltpu.VMEM((CK, h_), jnp.float32),
            pltpu.VMEM((CK, h_), jnp.float32),
            pltpu.VMEM((CK, h_), jnp.float32),
            pltpu.VMEM((CK,), jnp.float32),
            pltpu.VMEM((CK, h_), jnp.float32),
            pltpu.VMEM((CK,), jnp.float32),
            pltpu.VMEM((16,), jnp.float32),
            pltpu.VMEM_SHARED((n_pad, h_), jnp.float32),
            pltpu.VMEM_SHARED((n_pad,), jnp.float32),
            pltpu.SemaphoreType.DMA,
            pltpu.SemaphoreType.DMA,
            pltpu.SemaphoreType.DMA,
            pltpu.SemaphoreType.DMA,
            pltpu.SemaphoreType.DMA,
            pltpu.SemaphoreType.DMA,
            pltpu.SemaphoreType.DMA,
            pltpu.SemaphoreType.DMA,
        ],
        compiler_params=pltpu.CompilerParams(use_tc_tiling_on_sc=False,
                                             needs_layout_passes=False),
    )
    def conv_k(h_hbm, src_hbm, dst_hbm, beta_hbm, z16_hbm, z1_hbm,
               acc_out, den_out,
               sidx, didx, hsA, hdA, hsB, hdB, cA, dA, cB, dB, bvecv,
               acc_sh, den_sh,
               gsA, gdA, gsB, gdB, scA, sdA, scB, sdB):
        cid = lax.axis_index("c")
        sid = lax.axis_index("s")
        wid = sid * NC + cid

        r0 = sid * rows_per
        pltpu.sync_copy(z16_hbm.at[pl.ds(r0, rows_per)], acc_sh.at[pl.ds(r0, rows_per)])
        pltpu.sync_copy(z1_hbm.at[pl.ds(r0, rows_per)], den_sh.at[pl.ds(r0, rows_per)])

        pltpu.sync_copy(src_hbm.at[pl.ds(wid * ch, ch)], sidx)
        pltpu.sync_copy(dst_hbm.at[pl.ds(wid * ch, ch)], didx)
        pltpu.sync_copy(beta_hbm, bvecv)
        bvec = bvecv[...]
        plsc.subcore_barrier()

        rows0 = lax.iota(jnp.int32, 16)

        def compute(hs, hd, cb, db):
            for g in range(ng):
                rows = rows0 + (16 * g)
                acol = []
                dot = jnp.zeros((16,), jnp.float32)
                ns = jnp.zeros((16,), jnp.float32)
                nd = jnp.zeros((16,), jnp.float32)
                for f in range(h_):
                    cols = jnp.full((16,), f, jnp.int32)
                    a = plsc.load_gather(hs, (rows, cols))
                    b = plsc.load_gather(hd, (rows, cols))
                    acol.append(a)
                    dot += a * b
                    ns += a * a
                    nd += b * b
                r = _rsqrt16(jnp.maximum(ns * nd, 1e-30))
                s = jnp.exp(bvec * dot * r)
                for f in range(h_):
                    cols = jnp.full((16,), f, jnp.int32)
                    plsc.store_scatter(cb, (rows, cols), s * acol[f])
                db[pl.ds(16 * g, 16)] = s

        def fire(j, hs, hd, gs, gd):
            a = pltpu.async_copy(h_hbm.at[sidx.at[j]], hs, gs)
            b = pltpu.async_copy(h_hbm.at[didx.at[j]], hd, gd)
            return a, b

        def scat(j, cb, db, sc, sd):
            a = pltpu.async_copy(cb, acc_sh.at[didx.at[j]], sc, add=True)
            b = pltpu.async_copy(db, den_sh.at[didx.at[j]], sd, add=True)
            return a, b

        fire(0, hsA, hdA, gsA, gdA)

        def body(i, carry):
            j = 2 * i
            pltpu.make_async_copy(h_hbm.at[sidx.at[j]], hsA, gsA).wait()
            pltpu.make_async_copy(h_hbm.at[didx.at[j]], hdA, gdA).wait()
            fire(j + 1, hsB, hdB, gsB, gdB)
            @pl.when(i > 0)
            def _():
                pltpu.make_async_copy(cA, acc_sh.at[didx.at[j]], scA).wait()
                pltpu.make_async_copy(dA, den_sh.at[didx.at[j]], sdA).wait()
            compute(hsA, hdA, cA, dA)
            scat(j, cA, dA, scA, sdA)
            pltpu.make_async_copy(h_hbm.at[sidx.at[j + 1]], hsB, gsB).wait()
            pltpu.make_async_copy(h_hbm.at[didx.at[j + 1]], hdB, gdB).wait()
            @pl.when(i + 1 < ch // 2)
            def _():
                fire(j + 2, hsA, hdA, gsA, gdA)
            @pl.when(i > 0)
            def _():
                pltpu.make_async_copy(cB, acc_sh.at[didx.at[j]], scB).wait()
                pltpu.make_async_copy(dB, den_sh.at[didx.at[j]], sdB).wait()
            compute(hsB, hdB, cB, dB)
            scat(j + 1, cB, dB, scB, sdB)
            return carry

        lax.fori_loop(0, ch // 2, body, 0, unroll=False)
        pltpu.make_async_copy(cA, acc_sh.at[didx.at[0]], scA).wait()
        pltpu.make_async_copy(dA, den_sh.at[didx.at[0]], sdA).wait()
        pltpu.make_async_copy(cB, acc_sh.at[didx.at[0]], scB).wait()
        pltpu.make_async_copy(dB, den_sh.at[didx.at[0]], sdB).wait()
        plsc.subcore_barrier()

        def wb(k, carry):
            r = sid * rows_per + k * CK
            pltpu.sync_copy(acc_sh.at[pl.ds(r, CK)], cA)
            pltpu.sync_copy(cA, acc_out.at[cid].at[pl.ds(r, CK)])
            pltpu.sync_copy(den_sh.at[pl.ds(r, CK)], dA)
            pltpu.sync_copy(dA, den_out.at[cid].at[pl.ds(r, CK)])
            return carry

        lax.fori_loop(0, zc, wb, 0, unroll=False)

    return conv_k


def _combine_body(acc_ref, den_ref, o_ref):
    a = acc_ref[0] + acc_ref[1]
    d = jnp.maximum(den_ref[0] + den_ref[1], 1e-30)
    o_ref[...] = a / d


def _combine(acc, den):
    _, n_pad, h_ = acc.shape
    blk = 2048
    return pl.pallas_call(
        _combine_body,
        grid=(n_pad // blk,),
        in_specs=[
            pl.BlockSpec((2, blk, h_), lambda i: (0, i, 0)),
            pl.BlockSpec((2, blk, 1), lambda i: (0, i, 0)),
        ],
        out_specs=pl.BlockSpec((blk, h_), lambda i: (i, 0)),
        out_shape=jax.ShapeDtypeStruct((n_pad, h_), jnp.float32),
    )(acc, den)


def _final_body(acc_ref, den_ref, w_ref, b_ref, o_ref):
    a = acc_ref[0] + acc_ref[1]
    d = jnp.maximum(den_ref[0] + den_ref[1], 1e-30)
    h = a / d
    logits = jnp.dot(h, w_ref[...], preferred_element_type=jnp.float32) + b_ref[...]
    m = jnp.max(logits, axis=1, keepdims=True)
    lse = m + jnp.log(jnp.sum(jnp.exp(logits - m), axis=1, keepdims=True))
    o_ref[...] = logits - lse


def _final(acc, den, W4, b4, n):
    _, n_pad, h_ = acc.shape
    c = W4.shape[1]
    blk = 2000
    return pl.pallas_call(
        _final_body,
        grid=(n // blk,),
        in_specs=[
            pl.BlockSpec((2, blk, h_), lambda i: (0, i, 0)),
            pl.BlockSpec((2, blk, 1), lambda i: (0, i, 0)),
            pl.BlockSpec((h_, c), lambda i: (0, 0)),
            pl.BlockSpec((1, c), lambda i: (0, 0)),
        ],
        out_specs=pl.BlockSpec((blk, c), lambda i: (i, 0)),
        out_shape=jax.ShapeDtypeStruct((n, c), jnp.float32),
    )(acc, den, W4, b4.reshape(1, c))


def kernel(x, edge_index, W1, b1, beta2, beta3, W4, b4):
    n, d = x.shape
    h_ = W1.shape[1]
    e = edge_index.shape[1]

    n_pad = 51200
    etot = e + n
    sup = NW * CK
    ep = ((etot + sup - 1) // sup) * sup
    pad = ep - etot

    loops = jnp.arange(n, dtype=jnp.int32)
    junk = jnp.full((pad,), n, dtype=jnp.int32)
    src = jnp.concatenate([edge_index[0], loops, junk]).reshape(ep // CK, CK)
    dst = jnp.concatenate([edge_index[1], loops, junk]).reshape(ep // CK, CK)

    h1 = _dense1(x, W1, b1)
    h1p = jnp.concatenate([h1, jnp.zeros((n_pad - n, h_), jnp.float32)], axis=0)

    z16 = jnp.zeros((n_pad, h_), jnp.float32)
    z1 = jnp.zeros((n_pad,), jnp.float32)

    conv_k = _make_conv(n_pad, ep, h_)

    def conv(hp, beta):
        acc, den = conv_k(hp, src, dst, jnp.full((16,), beta, jnp.float32),
                          z16, z1)
        return acc, den.reshape(NC, n_pad, 1)

    acc1, den1 = conv(h1p, beta2)
    h2p = _combine(acc1, den1)
    acc2, den2 = conv(h2p, beta3)
    return _final(acc2, den2, W4, b4, n)


# speedup vs baseline: 48.0617x; 1.1139x over previous
import functools

import jax
import jax.numpy as jnp
from jax import lax
from jax.experimental import pallas as pl
from jax.experimental.pallas import tpu as pltpu, tpu_sc as plsc

NC = 2
NS = 16
NW = NC * NS
CK = 128


def _dense1_body(x_ref, w_ref, b_ref, o_ref):
    h = jnp.dot(x_ref[...], w_ref[...], preferred_element_type=jnp.float32)
    o_ref[...] = jnp.maximum(h + b_ref[...], 0.0)


def _dense1(x, W1, b1):
    n, d = x.shape
    h_ = W1.shape[1]
    blk = 2000
    return pl.pallas_call(
        _dense1_body,
        grid=(n // blk,),
        in_specs=[
            pl.BlockSpec((blk, d), lambda i: (i, 0)),
            pl.BlockSpec((d, h_), lambda i: (0, 0)),
            pl.BlockSpec((1, h_), lambda i: (0, 0)),
        ],
        out_specs=pl.BlockSpec((blk, h_), lambda i: (i, 0)),
        out_shape=jax.ShapeDtypeStruct((n, h_), jnp.float32),
    )(x, W1, b1.reshape(1, h_))


def _rsqrt16(v):
    i = plsc.bitcast(v, jnp.int32)
    y = plsc.bitcast(jnp.int32(0x5F3759DF) - (i >> 1), jnp.float32)
    for _ in range(3):
        y = y * (1.5 - 0.5 * v * y * y)
    return y


def _make_conv(n_pad, ep, h_):
    ch = ep // (NW * CK)
    rows_per = n_pad // NS
    zc = rows_per // CK
    ng = CK // 16
    mesh = plsc.VectorSubcoreMesh(core_axis_name="c", subcore_axis_name="s")

    @functools.partial(
        pl.kernel,
        mesh=mesh,
        out_type=(
            jax.ShapeDtypeStruct((NC, n_pad, h_), jnp.float32),
            jax.ShapeDtypeStruct((NC, n_pad), jnp.float32),
        ),
        scratch_types=[
            pltpu.VMEM((ch, CK), jnp.int32),
            pltpu.VMEM((ch, CK), jnp.int32),
            pltpu.VMEM((CK, h_), jnp.float32),
            pltpu.VMEM((CK, h_), jnp.float32),
            pltpu.VMEM((CK, h_), jnp.float32),
            pltpu.VMEM((CK, h_), jnp.float32),
            pltpu.VMEM((CK, h_), jnp.float32),
            pltpu.VMEM((CK,), jnp.float32),
            pltpu.VMEM((CK, h_), jnp.float32),
            pltpu.VMEM((CK,), jnp.float32),
            pltpu.VMEM((16,), jnp.float32),
            pltpu.VMEM_SHARED((n_pad, h_), jnp.float32),
            pltpu.VMEM_SHARED((n_pad,), jnp.float32),
            pltpu.SemaphoreType.DMA,
            pltpu.SemaphoreType.DMA,
            pltpu.SemaphoreType.DMA,
            pltpu.SemaphoreType.DMA,
            pltpu.SemaphoreType.DMA,
            pltpu.SemaphoreType.DMA,
            pltpu.SemaphoreType.DMA,
            pltpu.SemaphoreType.DMA,
        ],
        compiler_params=pltpu.CompilerParams(use_tc_tiling_on_sc=False,
                                             needs_layout_passes=False),
    )
    def conv_k(h_hbm, src_hbm, dst_hbm, beta_hbm, z16_hbm, z1_hbm,
               acc_out, den_out,
               sidx, didx, hsA, hdA, hsB, hdB, cA, dA, cB, dB, bvecv,
               acc_sh, den_sh,
               gsA, gdA, gsB, gdB, scA, sdA, scB, sdB):
        cid = lax.axis_index("c")
        sid = lax.axis_index("s")
        wid = sid * NC + cid

        r0 = sid * rows_per
        pltpu.sync_copy(z16_hbm.at[pl.ds(r0, rows_per)], acc_sh.at[pl.ds(r0, rows_per)])
        pltpu.sync_copy(z1_hbm.at[pl.ds(r0, rows_per)], den_sh.at[pl.ds(r0, rows_per)])

        pltpu.sync_copy(src_hbm.at[pl.ds(wid * ch, ch)], sidx)
        pltpu.sync_copy(dst_hbm.at[pl.ds(wid * ch, ch)], didx)
        pltpu.sync_copy(beta_hbm, bvecv)
        bvec = bvecv[...]
        plsc.subcore_barrier()

        rows0 = lax.iota(jnp.int32, 16)

        def compute(hs, hd, cb, db):
            for g in range(ng):
                rows = rows0 + (16 * g)
                acol = []
                dot = jnp.zeros((16,), jnp.float32)
                ns = jnp.zeros((16,), jnp.float32)
                nd = jnp.zeros((16,), jnp.float32)
                for f in range(h_):
                    cols = jnp.full((16,), f, jnp.int32)
                    a = plsc.load_gather(hs, (rows, cols))
                    b = plsc.load_gather(hd, (rows, cols))
                    acol.append(a)
                    dot += a * b
                    ns += a * a
                    nd += b * b
                r = _rsqrt16(jnp.maximum(ns * nd, 1e-30))
                s = jnp.exp(bvec * dot * r)
                for f in range(h_):
                    cols = jnp.full((16,), f, jnp.int32)
                    plsc.store_scatter(cb, (rows, cols), s * acol[f])
                db[pl.ds(16 * g, 16)] = s

        def fire(j, hs, hd, gs, gd):
            a = pltpu.async_copy(h_hbm.at[sidx.at[j]], hs, gs)
            b = pltpu.async_copy(h_hbm.at[didx.at[j]], hd, gd)
            return a, b

        def scat(j, cb, db, sc, sd):
            a = pltpu.async_copy(cb, acc_sh.at[didx.at[j]], sc, add=True)
            b = pltpu.async_copy(db, den_sh.at[didx.at[j]], sd, add=True)
            return a, b

        fire(0, hsA, hdA, gsA, gdA)

        def body(i, carry):
            j = 2 * i
            pltpu.make_async_copy(h_hbm.at[sidx.at[j]], hsA, gsA).wait()
            pltpu.make_async_copy(h_hbm.at[didx.at[j]], hdA, gdA).wait()
            fire(j + 1, hsB, hdB, gsB, gdB)
            @pl.when(i > 0)
            def _():
                pltpu.make_async_copy(cA, acc_sh.at[didx.at[j]], scA).wait()
                pltpu.make_async_copy(dA, den_sh.at[didx.at[j]], sdA).wait()
            compute(hsA, hdA, cA, dA)
            scat(j, cA, dA, scA, sdA)
            pltpu.make_async_copy(h_hbm.at[sidx.at[j + 1]], hsB, gsB).wait()
            pltpu.make_async_copy(h_hbm.at[didx.at[j + 1]], hdB, gdB).wait()
            @pl.when(i + 1 < ch // 2)
            def _():
                fire(j + 2, hsA, hdA, gsA, gdA)
            @pl.when(i > 0)
            def _():
                pltpu.make_async_copy(cB, acc_sh.at[didx.at[j]], scB).wait()
                pltpu.make_async_copy(dB, den_sh.at[didx.at[j]], sdB).wait()
            compute(hsB, hdB, cB, dB)
            scat(j + 1, cB, dB, scB, sdB)
            return carry

        lax.fori_loop(0, ch // 2, body, 0, unroll=False)
        pltpu.make_async_copy(cA, acc_sh.at[didx.at[0]], scA).wait()
        pltpu.make_async_copy(dA, den_sh.at[didx.at[0]], sdA).wait()
        pltpu.make_async_copy(cB, acc_sh.at[didx.at[0]], scB).wait()
        pltpu.make_async_copy(dB, den_sh.at[didx.at[0]], sdB).wait()
        plsc.subcore_barrier()

        def wb(k, carry):
            r = sid * rows_per + k * CK
            pltpu.sync_copy(acc_sh.at[pl.ds(r, CK)], cA)
            pltpu.sync_copy(cA, acc_out.at[cid].at[pl.ds(r, CK)])
            pltpu.sync_copy(den_sh.at[pl.ds(r, CK)], dA)
            pltpu.sync_copy(dA, den_out.at[cid].at[pl.ds(r, CK)])
            return carry

        lax.fori_loop(0, zc, wb, 0, unroll=False)

    return conv_k


def _make_sc_combine(n_pad, h_):
    rows_w = n_pad // NW
    cchunk = 160
    nch = rows_w // cchunk
    mesh = plsc.VectorSubcoreMesh(core_axis_name="c", subcore_axis_name="s")

    @functools.partial(
        pl.kernel,
        mesh=mesh,
        out_type=jax.ShapeDtypeStruct((n_pad, h_), jnp.float32),
        scratch_types=[
            pltpu.VMEM((cchunk, h_), jnp.float32),
            pltpu.VMEM((cchunk, h_), jnp.float32),
            pltpu.VMEM((cchunk,), jnp.float32),
            pltpu.VMEM((cchunk,), jnp.float32),
            pltpu.VMEM((cchunk, h_), jnp.float32),
        ],
        compiler_params=pltpu.CompilerParams(use_tc_tiling_on_sc=False,
                                             needs_layout_passes=False),
    )
    def combine_k(acc_hbm, den_hbm, h_out, a0v, a1v, d0v, d1v, hv):
        cid = lax.axis_index("c")
        sid = lax.axis_index("s")
        wid = sid * NC + cid
        r0 = wid * rows_w
        rows0 = lax.iota(jnp.int32, 16)

        def body(k, carry):
            rb = r0 + k * cchunk
            pltpu.sync_copy(acc_hbm.at[0].at[pl.ds(rb, cchunk)], a0v)
            pltpu.sync_copy(acc_hbm.at[1].at[pl.ds(rb, cchunk)], a1v)
            pltpu.sync_copy(den_hbm.at[0].at[pl.ds(rb, cchunk)], d0v)
            pltpu.sync_copy(den_hbm.at[1].at[pl.ds(rb, cchunk)], d1v)
            for g in range(cchunk // 16):
                rows = rows0 + 16 * g
                dsum = d0v[pl.ds(16 * g, 16)] + d1v[pl.ds(16 * g, 16)]
                rinv = 1.0 / jnp.maximum(dsum, 1e-30)
                for f in range(h_):
                    cols = jnp.full((16,), f, jnp.int32)
                    col = (plsc.load_gather(a0v, (rows, cols))
                           + plsc.load_gather(a1v, (rows, cols))) * rinv
                    plsc.store_scatter(hv, (rows, cols), col)
            pltpu.sync_copy(hv, h_out.at[pl.ds(rb, cchunk)])
            return carry

        lax.fori_loop(0, nch, body, 0, unroll=False)

    return combine_k


def _combine_body(acc_ref, den_ref, o_ref):
    a = acc_ref[0] + acc_ref[1]
    d = jnp.maximum(den_ref[0] + den_ref[1], 1e-30)
    o_ref[...] = a / d


def _combine(acc, den):
    _, n_pad, h_ = acc.shape
    blk = 2048
    return pl.pallas_call(
        _combine_body,
        grid=(n_pad // blk,),
        in_specs=[
            pl.BlockSpec((2, blk, h_), lambda i: (0, i, 0)),
            pl.BlockSpec((2, blk, 1), lambda i: (0, i, 0)),
        ],
        out_specs=pl.BlockSpec((blk, h_), lambda i: (i, 0)),
        out_shape=jax.ShapeDtypeStruct((n_pad, h_), jnp.float32),
    )(acc, den)


def _final_body(h_ref, w_ref, b_ref, o_ref):
    logits = jnp.dot(h_ref[...], w_ref[...], preferred_element_type=jnp.float32) + b_ref[...]
    m = jnp.max(logits, axis=1, keepdims=True)
    lse = m + jnp.log(jnp.sum(jnp.exp(logits - m), axis=1, keepdims=True))
    o_ref[...] = logits - lse


def _final(h3, W4, b4, n):
    _, h_ = h3.shape
    c = W4.shape[1]
    blk = 2000
    return pl.pallas_call(
        _final_body,
        grid=(n // blk,),
        in_specs=[
            pl.BlockSpec((blk, h_), lambda i: (i, 0)),
            pl.BlockSpec((h_, c), lambda i: (0, 0)),
            pl.BlockSpec((1, c), lambda i: (0, 0)),
        ],
        out_specs=pl.BlockSpec((blk, c), lambda i: (i, 0)),
        out_shape=jax.ShapeDtypeStruct((n, c), jnp.float32),
    )(h3, W4, b4.reshape(1, c))


def kernel(x, edge_index, W1, b1, beta2, beta3, W4, b4):
    n, d = x.shape
    h_ = W1.shape[1]
    e = edge_index.shape[1]

    n_pad = 51200
    etot = e + n
    sup = NW * CK
    ep = ((etot + sup - 1) // sup) * sup
    pad = ep - etot

    loops = jnp.arange(n, dtype=jnp.int32)
    junk = jnp.full((pad,), n, dtype=jnp.int32)
    src = jnp.concatenate([edge_index[0], loops, junk]).reshape(ep // CK, CK)
    dst = jnp.concatenate([edge_index[1], loops, junk]).reshape(ep // CK, CK)

    h1 = _dense1(x, W1, b1)
    h1p = jnp.concatenate([h1, jnp.zeros((n_pad - n, h_), jnp.float32)], axis=0)

    z16 = jnp.zeros((n_pad, h_), jnp.float32)
    z1 = jnp.zeros((n_pad,), jnp.float32)

    conv_k = _make_conv(n_pad, ep, h_)
    combine_k = _make_sc_combine(n_pad, h_)

    def conv(hp, beta):
        return conv_k(hp, src, dst, jnp.full((16,), beta, jnp.float32),
                      z16, z1)

    acc1, den1 = conv(h1p, beta2)
    h2p = combine_k(acc1, den1)
    acc2, den2 = conv(h2p, beta3)
    h3 = combine_k(acc2, den2)
    return _final(h3, W4, b4, n)
